# baseline (speedup 1.0000x reference)
import functools

import jax
import jax.numpy as jnp
from jax import lax
from jax.experimental import pallas as pl
from jax.experimental.pallas import tpu as pltpu
from jax.experimental.pallas import tpu_sc as plsc

B = 16384
AE = 50
NE = 20
NODE_V = 1000
CTX = 512
ST = 256
OUT_D = AE + CTX + AE + ST + NE
OFF_CTX = AE
OFF_PAR = AE + CTX
OFF_ST = OFF_PAR + AE
OFF_NODE = OFF_ST + ST
CTX_AL = OFF_CTX - (OFF_CTX % 8)
ST_AL = OFF_ST - (OFF_ST % 8)

NC, NS, L = 2, 16, 16
NW = NC * NS
ROWS_PER_W = B // NW
C = 64
NCHUNK = ROWS_PER_W // C


def _body(node_idx_hbm, prev_act_hbm, state_hbm, par_act_hbm, ctx_hbm,
          rule_hbm, token_hbm, node_tab_hbm, out_hbm,
          pa_v, qa_v, nidx_v, emb_p_v, emb_q_v, node_tab_v, out_v,
          sem_in, sem_g, sem_n):
  wid = lax.axis_index("s") * NC + lax.axis_index("c")
  base_w = wid * ROWS_PER_W
  iota = lax.iota(jnp.int32, L)
  zeros = jnp.zeros((L,), jnp.int32)

  nt = pltpu.async_copy(node_tab_hbm, node_tab_v, sem_n)

  def chunk(g, carry):
    row0 = base_w + g * C

    i1 = pltpu.async_copy(prev_act_hbm.at[pl.ds(row0, C), :], pa_v, sem_in)
    i2 = pltpu.async_copy(par_act_hbm.at[pl.ds(row0, C), :], qa_v, sem_in)
    i3 = pltpu.async_copy(node_idx_hbm.at[pl.ds(row0, C)], nidx_v, sem_in)
    d1 = pltpu.async_copy(ctx_hbm.at[pl.ds(row0, C), :],
                          out_v.at[:, pl.ds(CTX_AL, CTX)], sem_in)
    d2 = pltpu.async_copy(state_hbm.at[pl.ds(row0, C), :],
                          out_v.at[:, pl.ds(ST_AL, ST)], sem_in)
    i1.wait(); i2.wait(); i3.wait()

    def fetch_group(j, _):
      rows = iota + j * L
      tp_vec = plsc.load_gather(pa_v, [rows, zeros])
      vp_vec = plsc.load_gather(pa_v, [rows, zeros + 1])
      tq_vec = plsc.load_gather(qa_v, [rows, zeros])
      vq_vec = plsc.load_gather(qa_v, [rows, zeros + 1])
      for l in range(L):
        i = j * L + l
        tp, vp = tp_vec[l], vp_vec[l]
        tq, vq = tq_vec[l], vq_vec[l]

        @pl.when(tp == 0)
        def _():
          pltpu.async_copy(rule_hbm.at[pl.ds(vp, 1), :],
                           emb_p_v.at[pl.ds(i, 1), :], sem_g)

        @pl.when(tp != 0)
        def _():
          pltpu.async_copy(token_hbm.at[pl.ds(vp, 1), :],
                           emb_p_v.at[pl.ds(i, 1), :], sem_g)

        @pl.when(tq == 0)
        def _():
          pltpu.async_copy(rule_hbm.at[pl.ds(vq, 1), :],
                           emb_q_v.at[pl.ds(i, 1), :], sem_g)

        @pl.when(tq != 0)
        def _():
          pltpu.async_copy(token_hbm.at[pl.ds(vq, 1), :],
                           emb_q_v.at[pl.ds(i, 1), :], sem_g)

      return 0

    lax.fori_loop(0, C // L, fetch_group, 0)

    d1.wait(); d2.wait()

    def shift_row(i, _):
      for k in reversed(range(CTX // L)):
        v = out_v[i, pl.ds(CTX_AL + k * L, L)]
        out_v[i, pl.ds(OFF_CTX + k * L, L)] = v
      for k in reversed(range(ST // L)):
        v = out_v[i, pl.ds(ST_AL + k * L, L)]
        out_v[i, pl.ds(OFF_ST + k * L, L)] = v
      return 0

    lax.fori_loop(0, C, shift_row, 0)

    pltpu.make_async_copy(rule_hbm.at[pl.ds(0, C), :], emb_p_v, sem_g).wait()
    pltpu.make_async_copy(rule_hbm.at[pl.ds(0, C), :], emb_q_v, sem_g).wait()

    def place_col(c, _):
      cvec = zeros + c
      for j in range(C // L):
        rows = iota + (j * L)
        vp = plsc.load_gather(emb_p_v, [rows, cvec])
        plsc.store_scatter(out_v, [rows, cvec], vp)
        vq = plsc.load_gather(emb_q_v, [rows, cvec])
        plsc.store_scatter(out_v, [rows, cvec + OFF_PAR], vq)
      return 0

    lax.fori_loop(0, AE, place_col, 0)

    def node_col(c, _):
      cvec = zeros + c
      for j in range(C // L):
        rows = iota + (j * L)
        ids = nidx_v[pl.ds(j * L, L)]
        v = plsc.load_gather(node_tab_v, [ids, cvec])
        plsc.store_scatter(out_v, [rows, cvec + OFF_NODE], v)
      return 0

    lax.fori_loop(0, NE, node_col, 0)

    pltpu.sync_copy(out_v, out_hbm.at[pl.ds(row0, C), :])
    return carry

  nt.wait()
  lax.fori_loop(0, NCHUNK, chunk, 0)


@jax.jit
def _lstm_input(current_node_type, previous_action, parent_state,
                parent_action, context, rule_table, token_table, node_table):
  mesh = plsc.VectorSubcoreMesh(core_axis_name="c", subcore_axis_name="s",
                                num_cores=NC, num_subcores=NS)
  f = functools.partial(
      pl.kernel,
      out_type=jax.ShapeDtypeStruct((B, OUT_D), jnp.float32),
      mesh=mesh,
      scratch_types=[
          pltpu.VMEM((C, 2), jnp.int32),
          pltpu.VMEM((C, 2), jnp.int32),
          pltpu.VMEM((C,), jnp.int32),
          pltpu.VMEM((C, AE), jnp.float32),
          pltpu.VMEM((C, AE), jnp.float32),
          pltpu.VMEM((NODE_V, NE), jnp.float32),
          pltpu.VMEM((C, OUT_D), jnp.float32),
          pltpu.SemaphoreType.DMA,
          pltpu.SemaphoreType.DMA,
          pltpu.SemaphoreType.DMA,
      ],
      compiler_params=pltpu.CompilerParams(use_tc_tiling_on_sc=False,
                                           needs_layout_passes=False),
  )(_body)
  return f(current_node_type, previous_action, parent_state, parent_action,
           context, rule_table, token_table, node_table)


def kernel(current_node_type, previous_action, parent_state, parent_action,
           context, rule_embedding_table, token_embedding_table,
           node_embedding_table):
  return _lstm_input(
      current_node_type.astype(jnp.int32),
      previous_action.astype(jnp.int32),
      parent_state,
      parent_action.astype(jnp.int32),
      context,
      rule_embedding_table,
      token_embedding_table,
      node_embedding_table)


# speedup vs baseline: 1.0219x; 1.0219x over previous
import functools

import jax
import jax.numpy as jnp
from jax import lax
from jax.experimental import pallas as pl
from jax.experimental.pallas import tpu as pltpu
from jax.experimental.pallas import tpu_sc as plsc

B = 16384
AE = 50
AEP = 56
NE = 20
NEP = 24
NODE_V = 1000
RULE_V = 100000
CTX = 512
ST = 256
OUT_D = AE + CTX + AE + ST + NE
OFF_CTX = AE
OFF_PAR = AE + CTX
OFF_ST = OFF_PAR + AE
OFF_NODE = OFF_ST + ST

NC, NS, L = 2, 16, 16
NW = NC * NS
ROWS_PER_W = B // NW
C = 32
NCHUNK = ROWS_PER_W // C
TR = C // 8
CTX_TC = CTX // 128
ST_TC = ST // 128


def _body(node_idx_hbm, act_p_hbm, st_hbm, act_q_hbm, ctx_hbm,
          rule_hbm, token_hbm, ntab_hbm, out_hbm,
          tp_v, vp_v, tq_v, vq_v, nidx_v, emb_p_v, emb_q_v, ntab_v,
          ctx_s, st_s, out_v, sem_in, sem_g, sem_n):
  wid = lax.axis_index("s") * NC + lax.axis_index("c")
  base_w = wid * ROWS_PER_W
  iota = lax.iota(jnp.int32, L)

  nt = pltpu.async_copy(ntab_hbm, ntab_v, sem_n)

  def chunk(g, carry):
    row0 = base_w + g * C

    i1 = pltpu.async_copy(act_p_hbm.at[pl.ds(row0, C)], tp_v, sem_in)
    i2 = pltpu.async_copy(act_p_hbm.at[pl.ds(B + row0, C)], vp_v, sem_in)
    i3 = pltpu.async_copy(act_q_hbm.at[pl.ds(row0, C)], tq_v, sem_in)
    i4 = pltpu.async_copy(act_q_hbm.at[pl.ds(B + row0, C)], vq_v, sem_in)
    i5 = pltpu.async_copy(node_idx_hbm.at[pl.ds(row0, C)], nidx_v, sem_in)
    d1 = pltpu.async_copy(ctx_hbm.at[pl.ds(row0 // 8, TR)], ctx_s, sem_in)
    d2 = pltpu.async_copy(st_hbm.at[pl.ds(row0 // 8, TR)], st_s, sem_in)
    i1.wait(); i2.wait(); i3.wait(); i4.wait(); i5.wait()

    def fetch_group(j, _):
      tp_vec = tp_v[pl.ds(j * L, L)]
      vp_vec = vp_v[pl.ds(j * L, L)]
      tq_vec = tq_v[pl.ds(j * L, L)]
      vq_vec = vq_v[pl.ds(j * L, L)]
      for l in range(L):
        i = j * L + l
        tp, vp = tp_vec[l], vp_vec[l]
        tq, vq = tq_vec[l], vq_vec[l]

        @pl.when(tp == 0)
        def _():
          pltpu.async_copy(rule_hbm.at[pl.ds(vp * AEP, AEP)],
                           emb_p_v.at[pl.ds(i * AEP, AEP)], sem_g)

        @pl.when(tp != 0)
        def _():
          pltpu.async_copy(token_hbm.at[pl.ds(vp * AEP, AEP)],
                           emb_p_v.at[pl.ds(i * AEP, AEP)], sem_g)

        @pl.when(tq == 0)
        def _():
          pltpu.async_copy(rule_hbm.at[pl.ds(vq * AEP, AEP)],
                           emb_q_v.at[pl.ds(i * AEP, AEP)], sem_g)

        @pl.when(tq != 0)
        def _():
          pltpu.async_copy(token_hbm.at[pl.ds(vq * AEP, AEP)],
                           emb_q_v.at[pl.ds(i * AEP, AEP)], sem_g)

      return 0

    lax.fori_loop(0, C // L, fetch_group, 0)

    d1.wait(); d2.wait()
    for tr in range(TR):
      for sl in range(8):
        r = tr * 8 + sl
        for tc in range(CTX_TC):
          for k in range(128 // L):
            v = ctx_s[tr, tc, sl, pl.ds(k * L, L)]
            out_v[r, pl.ds(OFF_CTX + tc * 128 + k * L, L)] = v
        for tc in range(ST_TC):
          for k in range(128 // L):
            v = st_s[tr, tc, sl, pl.ds(k * L, L)]
            out_v[r, pl.ds(OFF_ST + tc * 128 + k * L, L)] = v

    def node_col(c, _):
      for j in range(C // L):
        rows = iota + (j * L)
        ids = nidx_v[pl.ds(j * L, L)]
        v = plsc.load_gather(ntab_v, [ids * NEP + c])
        plsc.store_scatter(out_v, [rows, iota * 0 + (c + OFF_NODE)], v)
      return 0

    lax.fori_loop(0, NE, node_col, 0)

    pltpu.make_async_copy(rule_hbm.at[pl.ds(0, C * AEP)], emb_p_v,
                          sem_g).wait()
    pltpu.make_async_copy(rule_hbm.at[pl.ds(0, C * AEP)], emb_q_v,
                          sem_g).wait()

    def place_col(c, _):
      for j in range(C // L):
        rows = iota + (j * L)
        flat = rows * AEP + c
        vp = plsc.load_gather(emb_p_v, [flat])
        plsc.store_scatter(out_v, [rows, iota * 0 + c], vp)
        vq = plsc.load_gather(emb_q_v, [flat])
        plsc.store_scatter(out_v, [rows, iota * 0 + (c + OFF_PAR)], vq)
      return 0

    lax.fori_loop(0, AE, place_col, 0)

    pltpu.sync_copy(out_v, out_hbm.at[pl.ds(row0, C), :])
    return carry

  nt.wait()
  lax.fori_loop(0, NCHUNK, chunk, 0)


@jax.jit
def _lstm_input(node_idx, act_p, st4, act_q, ctx4, rule_flat, token_flat,
                ntab_flat):
  mesh = plsc.VectorSubcoreMesh(core_axis_name="c", subcore_axis_name="s",
                                num_cores=NC, num_subcores=NS)
  f = functools.partial(
      pl.kernel,
      out_type=jax.ShapeDtypeStruct((B, OUT_D), jnp.float32),
      mesh=mesh,
      scratch_types=[
          pltpu.VMEM((C,), jnp.int32),
          pltpu.VMEM((C,), jnp.int32),
          pltpu.VMEM((C,), jnp.int32),
          pltpu.VMEM((C,), jnp.int32),
          pltpu.VMEM((C,), jnp.int32),
          pltpu.VMEM((C * AEP,), jnp.float32),
          pltpu.VMEM((C * AEP,), jnp.float32),
          pltpu.VMEM((NODE_V * NEP,), jnp.float32),
          pltpu.VMEM((TR, CTX_TC, 8, 128), jnp.float32),
          pltpu.VMEM((TR, ST_TC, 8, 128), jnp.float32),
          pltpu.VMEM((C, OUT_D), jnp.float32),
          pltpu.SemaphoreType.DMA,
          pltpu.SemaphoreType.DMA,
          pltpu.SemaphoreType.DMA,
      ],
      compiler_params=pltpu.CompilerParams(use_tc_tiling_on_sc=False,
                                           needs_layout_passes=False),
  )(_body)
  return f(node_idx, act_p, st4, act_q, ctx4, rule_flat, token_flat,
           ntab_flat)


def kernel(current_node_type, previous_action, parent_state, parent_action,
           context, rule_embedding_table, token_embedding_table,
           node_embedding_table):
  act_p = previous_action.astype(jnp.int32).T.reshape(-1)
  act_q = parent_action.astype(jnp.int32).T.reshape(-1)
  ctx4 = context.reshape(B // 8, 8, CTX // 128, 128).transpose(0, 2, 1, 3)
  st4 = parent_state.reshape(B // 8, 8, ST // 128, 128).transpose(0, 2, 1, 3)
  rule_flat = jnp.pad(rule_embedding_table, ((0, 0), (0, AEP - AE))).reshape(-1)
  token_flat = jnp.pad(token_embedding_table, ((0, 0), (0, AEP - AE))).reshape(-1)
  ntab_flat = jnp.pad(node_embedding_table, ((0, 0), (0, NEP - NE))).reshape(-1)
  return _lstm_input(current_node_type.astype(jnp.int32), act_p, st4, act_q,
                     ctx4, rule_flat, token_flat, ntab_flat)
